# baseline (speedup 1.0000x reference)
import functools

import jax
import jax.numpy as jnp
from jax import lax
from jax.experimental import pallas as pl
from jax.experimental.pallas import tpu as pltpu
from jax.experimental.pallas import tpu_sc as plsc

_NC = 2
_NS = 16
_NW = _NC * _NS
_CH_ROWS = 5000
_ONES_CH = 10000
_UNROLL = 5


def _make_sc_kernel(e):
    rows_w = e // _NW
    nch = rows_w // _CH_ROWS
    vecs = (_CH_ROWS * 2) // 16

    mesh = plsc.VectorSubcoreMesh(core_axis_name="c", subcore_axis_name="s")

    @functools.partial(
        pl.kernel,
        out_type=[
            jax.ShapeDtypeStruct((e,), jnp.int32),
            jax.ShapeDtypeStruct((_NC, 16), jnp.int32),
        ],
        mesh=mesh,
        compiler_params=pltpu.CompilerParams(
            needs_layout_passes=False, use_tc_tiling_on_sc=False),
        scratch_types=[
            pltpu.VMEM((_CH_ROWS, 2), jnp.int32),
            pltpu.VMEM((_CH_ROWS, 2), jnp.int32),
            pltpu.VMEM((_ONES_CH,), jnp.int32),
            pltpu.VMEM((16,), jnp.int32),
            pltpu.VMEM((_NS, 16), jnp.int32),
            pltpu.VMEM_SHARED((_NS, 16), jnp.int32),
            pltpu.SemaphoreType.DMA,
            pltpu.SemaphoreType.DMA,
            pltpu.SemaphoreType.DMA,
        ],
    )
    def sc_kernel(x_hbm, vals_hbm, pmax_hbm,
                  buf0, buf1, ones_buf, vbuf, stage, shared,
                  sem_a, sem_b, sem_ones):
        c = lax.axis_index("c")
        s = lax.axis_index("s")
        wid = s * _NC + c
        base = wid * rows_w

        ones_vec = jnp.ones((16,), jnp.int32)

        def fill(i, carry):
            ones_buf[pl.ds(i * 16, 16)] = ones_vec
            return carry

        lax.fori_loop(0, _ONES_CH // 16, fill, 0)

        nones = rows_w // _ONES_CH
        ones_copies = [
            pltpu.make_async_copy(
                ones_buf,
                vals_hbm.at[pl.ds(base + j * _ONES_CH, _ONES_CH)],
                sem_ones,
            )
            for j in range(nones)
        ]
        for cp in ones_copies:
            cp.start()

        bufs = (buf0, buf1)
        in_sems = (sem_a, sem_b)

        def in_copy(j):
            return pltpu.make_async_copy(
                x_hbm.at[pl.ds(base + j * _CH_ROWS, _CH_ROWS)],
                bufs[j % 2], in_sems[j % 2])

        iota = lax.iota(jnp.int32, 16)
        base_rows = lax.shift_right_logical(iota, 1)
        col_idx = jnp.bitwise_and(iota, 1)
        neg_inf = jnp.full((16,), jnp.iinfo(jnp.int32).min, jnp.int32)

        def chunk_max(b, accs):
            def body(k, accs_in):
                a0, a1 = accs_in
                r0 = base_rows + k * (_UNROLL * 8)
                loc = [
                    plsc.load_gather(b, [r0 + t * 8, col_idx])
                    for t in range(_UNROLL)
                ]
                m = [loc[0], loc[1]]
                for t in range(2, _UNROLL):
                    m[t % 2] = jnp.maximum(m[t % 2], loc[t])
                return (jnp.maximum(a0, m[0]), jnp.maximum(a1, m[1]))

            return lax.fori_loop(0, vecs // _UNROLL, body, accs)

        accs = (neg_inf, neg_inf)
        in_copy(0).start()
        for j in range(nch):
            if j + 1 < nch:
                in_copy(j + 1).start()
            in_copy(j).wait()
            accs = chunk_max(bufs[j % 2], accs)
        for cp in ones_copies:
            cp.wait()

        vbuf[...] = jnp.maximum(accs[0], accs[1])
        pltpu.sync_copy(vbuf, shared.at[s])
        plsc.subcore_barrier()

        @pl.when(s == 0)
        def _reduce():
            pltpu.sync_copy(shared, stage)
            m = stage[0]
            for i in range(1, _NS):
                m = jnp.maximum(m, stage[i])
            vbuf[...] = m
            pltpu.sync_copy(vbuf, pmax_hbm.at[c])

    return sc_kernel


def _finish_body(p_ref, nmax_ref):
    nmax_ref[0, 0] = jnp.max(p_ref[...]) + 1


def kernel(edge_indices):
    ei2 = jnp.reshape(edge_indices, (-1, 2))
    e = ei2.shape[0]

    vals, pmax = _make_sc_kernel(e)(ei2)

    nmax = pl.pallas_call(
        _finish_body,
        out_specs=pl.BlockSpec(memory_space=pltpu.SMEM),
        out_shape=jax.ShapeDtypeStruct((1, 1), jnp.int32),
    )(pmax)

    ei_out = ei2.astype(jnp.int64)
    vals_out = vals.astype(jnp.int64)
    n_nodes = nmax[0, 0].astype(jnp.int64)
    return (ei_out, vals_out, n_nodes)


# speedup vs baseline: 4.2229x; 4.2229x over previous
import functools

import jax
import jax.numpy as jnp
from jax import lax
from jax.experimental import pallas as pl
from jax.experimental.pallas import tpu as pltpu
from jax.experimental.pallas import tpu_sc as plsc

_NC = 2
_NS = 16
_NW = _NC * _NS
_ONES_CH = 10000
_GRID = 50
_BLK = 32000


def _make_sc_ones(e):
    per_w = e // _NW
    nones = per_w // _ONES_CH
    mesh = plsc.VectorSubcoreMesh(core_axis_name="c", subcore_axis_name="s")

    @functools.partial(
        pl.kernel,
        out_type=jax.ShapeDtypeStruct((e,), jnp.int32),
        mesh=mesh,
        scratch_types=[
            pltpu.VMEM((_ONES_CH,), jnp.int32),
            pltpu.SemaphoreType.DMA,
        ],
    )
    def sc_ones(vals_hbm, ones_buf, sem):
        c = lax.axis_index("c")
        s = lax.axis_index("s")
        wid = s * _NC + c
        base = wid * per_w

        ones_vec = jnp.ones((16,), jnp.int32)

        def fill(i, carry):
            ones_buf[pl.ds(i * 16, 16)] = ones_vec
            return carry

        lax.fori_loop(0, _ONES_CH // 16, fill, 0)

        copies = [
            pltpu.make_async_copy(
                ones_buf, vals_hbm.at[pl.ds(base + j * _ONES_CH, _ONES_CH)], sem)
            for j in range(nones)
        ]
        for cp in copies:
            cp.start()
        for cp in copies:
            cp.wait()

    return sc_ones


def _max_body(x_ref, nmax_ref):
    i = pl.program_id(0)
    m = jnp.max(x_ref[...])
    prev = jnp.where(i == 0, jnp.iinfo(jnp.int32).min, nmax_ref[0, 0])
    cur = jnp.maximum(prev, m)
    nmax_ref[0, 0] = jnp.where(i == pl.num_programs(0) - 1, cur + 1, cur)


def kernel(edge_indices):
    ei2 = jnp.reshape(edge_indices, (-1, 2))
    e = ei2.shape[0]

    vals = _make_sc_ones(e)()

    nmax = pl.pallas_call(
        _max_body,
        grid=(_GRID,),
        in_specs=[pl.BlockSpec((_BLK, 2), lambda i: (i, 0))],
        out_specs=pl.BlockSpec(
            memory_space=pltpu.SMEM, block_shape=(1, 1), index_map=lambda i: (0, 0)),
        out_shape=jax.ShapeDtypeStruct((1, 1), jnp.int32),
    )(ei2)

    ei_out = ei2.astype(jnp.int64)
    vals_out = vals.astype(jnp.int64)
    n_nodes = nmax[0, 0].astype(jnp.int64)
    return (ei_out, vals_out, n_nodes)
